# baseline (speedup 1.0000x reference)
import functools

import jax
import jax.numpy as jnp
from jax import lax
from jax.experimental import pallas as pl
from jax.experimental.pallas import tpu as pltpu
from jax.experimental.pallas import tpu_sc as plsc

NW = 32
L = 16


@functools.lru_cache(maxsize=None)
def _build(B, S, D, V):
    SPW = B // NW
    C = 8
    NCHUNK = SPW // C
    CHW = C * S
    GB = 64
    NBLK = CHW // GB

    mesh = plsc.VectorSubcoreMesh(core_axis_name="c", subcore_axis_name="s")

    @functools.partial(
        pl.kernel,
        out_type=jax.ShapeDtypeStruct((B * D,), jnp.float32),
        mesh=mesh,
        scratch_types=[
            pltpu.VMEM((CHW,), jnp.int32),
            pltpu.VMEM((CHW,), jnp.int32),
            pltpu.VMEM((NBLK, GB), jnp.int32),
            pltpu.VMEM((CHW, D), jnp.float32),
            pltpu.VMEM((C * D,), jnp.float32),
            pltpu.VMEM((D,), jnp.float32),
            pltpu.SemaphoreType.DMA,
        ],
        compiler_params=pltpu.CompilerParams(
            use_tc_tiling_on_sc=False, needs_layout_passes=False),
    )
    def enc(tok_hbm, mask_hbm, w0_hbm, table_hbm, out_hbm,
            tok_v, mask_v, idx_v, rows_v, outb_v, w0_v, sem):
        cid = lax.axis_index("c")
        sid = lax.axis_index("s")
        wid = sid * 2 + cid

        pltpu.sync_copy(w0_hbm, w0_v)
        w0a = w0_v[pl.ds(0, L)]
        w0b = w0_v[pl.ds(L, L)]
        lanes = lax.iota(jnp.int32, L)
        tailm = jnp.minimum(jnp.maximum(lanes - (L - S % L - 1), 0), 1)

        def chunk_body(ci, carry):
            tbase = (wid * SPW + ci * C) * S
            pltpu.sync_copy(tok_hbm.at[pl.ds(tbase, CHW)], tok_v)
            pltpu.sync_copy(mask_hbm.at[pl.ds(tbase, CHW)], mask_v)

            def idx_body(bb, c2):
                for u in range(GB // L):
                    off = bb * GB + u * L
                    t = tok_v[pl.ds(off, L)]
                    m = mask_v[pl.ds(off, L)]
                    idx_v[bb, pl.ds(u * L, L)] = t * m
                return c2
            lax.fori_loop(0, NBLK, idx_body, 0)

            def fire(bb, c2):
                pltpu.make_async_copy(
                    table_hbm.at[idx_v.at[bb]],
                    rows_v.at[pl.ds(bb * GB, GB)], sem).start()
                return c2
            lax.fori_loop(0, NBLK, fire, 0)

            def drain(bb, c2):
                pltpu.make_async_copy(
                    table_hbm.at[idx_v.at[0]],
                    rows_v.at[pl.ds(0, GB)], sem).wait()
                return c2
            lax.fori_loop(0, NBLK, drain, 0)

            def samp_body(s, c2):
                mb = s * S
                cnt_vec = mask_v[pl.ds(mb, L)]
                for q in range(1, S // L):
                    cnt_vec = cnt_vec + mask_v[pl.ds(mb + q * L, L)]
                cnt_vec = cnt_vec + mask_v[pl.ds(mb + S - L, L)] * tailm
                cnt = jnp.sum(cnt_vec)

                acc0 = jnp.zeros((L,), jnp.float32)
                acc1 = jnp.zeros((L,), jnp.float32)
                for j in range(S):
                    acc0 = acc0 + rows_v[mb + j, pl.ds(0, L)]
                    acc1 = acc1 + rows_v[mb + j, pl.ds(L, L)]

                cntf = jnp.full((L,), cnt.astype(jnp.float32))
                n0 = jnp.float32(S) - cntf
                scale = jnp.float32(1.0) / jnp.maximum(cntf, 1.0)
                outb_v[pl.ds(s * D, L)] = (acc0 - n0 * w0a) * scale
                outb_v[pl.ds(s * D + L, L)] = (acc1 - n0 * w0b) * scale
                return c2
            lax.fori_loop(0, C, samp_body, 0)

            pltpu.sync_copy(
                outb_v, out_hbm.at[pl.ds((wid * SPW + ci * C) * D, C * D)])
            return carry
        lax.fori_loop(0, NCHUNK, chunk_body, 0)

    return enc


def kernel(tokens, mask, W):
    B, S = tokens.shape
    V, D = W.shape
    enc = _build(B, S, D, V)
    out = enc(tokens.reshape(-1),
              mask.astype(jnp.int32).reshape(-1),
              W[0],
              W)
    return out.reshape(B, D)
